# baseline (speedup 1.0000x reference)
import functools

import numpy as np

import jax
import jax.numpy as jnp
from jax import lax
from jax.experimental import pallas as pl
from jax.experimental.pallas import tpu as pltpu
from jax.experimental.pallas import tpu_sc as plsc

N = 10000
E = 320000
D = 128
DE = 16
B = 1024

NC = 2
NS = 16
NW = NC * NS
EPW = E // NW
SUB = 80
SUBN = 1
C = SUB * SUBN
NCHUNK = EPW // C
RPT = N // NS
BPT = B // NS


def _xa_body(x_ref, w_ref, o_ref):
    o_ref[...] = jnp.dot(x_ref[...], w_ref[...],
                         preferred_element_type=jnp.float32)


def _eb_body(a_ref, w_ref, b_ref, o_ref):
    m = (jnp.dot(a_ref[...], w_ref[...],
                 preferred_element_type=jnp.float32) + b_ref[...])
    bits = jax.lax.bitcast_convert_type(m, jnp.int32) + jnp.int32(0x8000)
    lo = jax.lax.shift_right_logical(bits[:, :D // 2], 16)
    hi = jnp.bitwise_and(bits[:, D // 2:], jnp.int32(-65536))
    o_ref[...] = jnp.bitwise_or(lo, hi)


_EB_R = 8000


def _tc_pre(x, attr, W1, W2, b_msg):
    xa = pl.pallas_call(
        _xa_body,
        out_shape=jax.ShapeDtypeStruct((N, D), jnp.float32),
    )(x, W1)
    eb = pl.pallas_call(
        _eb_body,
        grid=(E // _EB_R,),
        in_specs=[
            pl.BlockSpec((_EB_R, DE), lambda i: (i, 0)),
            pl.BlockSpec((DE, D), lambda i: (0, 0)),
            pl.BlockSpec((1, D), lambda i: (0, 0)),
        ],
        out_specs=pl.BlockSpec((_EB_R, D // 2), lambda i: (i, 0)),
        out_shape=jax.ShapeDtypeStruct((E, D // 2), jnp.int32),
    )(attr, W2, b_msg.reshape(1, D))
    return xa, eb


def _sc_body(xa_hbm, eb_hbm, src_hbm, dst_hbm, idx_hbm, x_hbm, zeros_hbm,
             aggsel_hbm, xsel_hbm,
             agg_sh, src_v, dst_v, e_v0, e_v1, g_v, idx_v,
             sem_l0, sem_l1, sem_g0, sem_g1, sem_s0, sem_s1):
    cid = lax.axis_index("c")
    sid = lax.axis_index("s")
    wid = cid * NS + sid
    e_v = (e_v0, e_v1)
    sem_l = (sem_l0, sem_l1)
    sem_g = (sem_g0, sem_g1)
    sem_s = (sem_s0, sem_s1)

    def issue_loads(k, s):
        crow = wid * NCHUNK + k
        ebase = wid * EPW + k * C
        pltpu.async_copy(src_hbm.at[crow], src_v.at[s], sem_l[s])
        pltpu.async_copy(dst_hbm.at[crow], dst_v.at[s], sem_l[s])
        eoff = pl.multiple_of(ebase * (D // 2), C * D // 2)
        pltpu.async_copy(eb_hbm.at[pl.ds(eoff, C * D // 2)],
                         e_v[s], sem_l[s])

    def wait_loads(s):
        pltpu.make_async_copy(src_hbm.at[0], src_v.at[s],
                              sem_l[s]).wait()
        pltpu.make_async_copy(dst_hbm.at[0], dst_v.at[s],
                              sem_l[s]).wait()
        pltpu.make_async_copy(eb_hbm.at[pl.ds(0, C * D // 2)], e_v[s],
                              sem_l[s]).wait()

    def issue_gather(s):
        for j in range(SUBN):
            pltpu.async_copy(xa_hbm.at[src_v.at[s, j]],
                             g_v.at[s, pl.ds(j * SUB, SUB)], sem_g[s])

    def wait_gather(s):
        pltpu.make_async_copy(xa_hbm.at[pl.ds(0, C)], g_v.at[s],
                              sem_g[s]).wait()

    def issue_scatter(s):
        for j in range(SUBN):
            pltpu.async_copy(g_v.at[s, pl.ds(j * SUB, SUB)],
                             agg_sh.at[dst_v.at[s, j]], sem_s[s], add=True)

    def wait_scatter(s):
        pltpu.make_async_copy(xa_hbm.at[pl.ds(0, C)], g_v.at[s],
                              sem_s[s]).wait()

    issue_loads(0, 0)
    issue_loads(1, 1)
    pltpu.sync_copy(zeros_hbm, agg_sh.at[pl.ds(sid * RPT, RPT)])
    plsc.subcore_barrier()

    wait_loads(0)
    issue_gather(0)

    def pipeline_step(i, b):
        o = 1 - b

        @pl.when(i + 1 < NCHUNK)
        def _():
            wait_loads(o)

            @pl.when(i >= 1)
            def _():
                wait_scatter(o)

            issue_gather(o)

        wait_gather(b)

        @plsc.parallel_loop(0, C, step=1, unroll=4)
        def _(r):
            for k2 in range(D // 32):
                off = pl.multiple_of(r * (D // 2) + 16 * k2, 16)
                w = e_v[b][pl.ds(off, 16)]
                ea = plsc.bitcast(jnp.left_shift(w, 16), jnp.float32)
                eo = plsc.bitcast(jnp.bitwise_and(w, jnp.int32(-65536)),
                                  jnp.float32)
                sl0 = pl.ds(32 * k2, 16)
                sl1 = pl.ds(32 * k2 + 16, 16)
                g_v[b, r, sl0] = jnp.maximum(g_v[b, r, sl0] + ea, 0.0)
                g_v[b, r, sl1] = jnp.maximum(g_v[b, r, sl1] + eo, 0.0)

        issue_scatter(b)

        @pl.when(i + 2 < NCHUNK)
        def _():
            issue_loads(i + 2, b)

    def outer(t, carry):
        pipeline_step(2 * t, 0)
        pipeline_step(2 * t + 1, 1)
        return carry

    lax.fori_loop(0, NCHUNK // 2, outer, 0)
    pipeline_step(jnp.int32(NCHUNK - 1), 0)
    wait_scatter(0)
    wait_scatter(1)
    plsc.subcore_barrier()

    pltpu.sync_copy(idx_hbm.at[pl.ds(sid * BPT, BPT)], idx_v)
    pltpu.async_copy(agg_sh.at[idx_v], g_v.at[0, pl.ds(0, BPT)], sem_g0).wait()
    pltpu.sync_copy(g_v.at[0, pl.ds(0, BPT)],
                    aggsel_hbm.at[cid, pl.ds(sid * BPT, BPT)])

    @pl.when(cid == 0)
    def _():
        pltpu.async_copy(x_hbm.at[idx_v], g_v.at[1, pl.ds(0, BPT)],
                         sem_g1).wait()
        pltpu.sync_copy(g_v.at[1, pl.ds(0, BPT)],
                        xsel_hbm.at[pl.ds(sid * BPT, BPT)])


def _sc_call(xa, eb, src, dst, idx, x, zeros):
    mesh = plsc.VectorSubcoreMesh(core_axis_name="c", subcore_axis_name="s",
                                  num_cores=NC, num_subcores=NS)
    f = pl.kernel(
        _sc_body,
        out_type=(jax.ShapeDtypeStruct((NC, B, D), jnp.float32),
                  jax.ShapeDtypeStruct((B, D), jnp.float32)),
        mesh=mesh,
        compiler_params=pltpu.CompilerParams(needs_layout_passes=False),
        scratch_types=[
            pltpu.VMEM_SHARED((N, D), jnp.float32),
            pltpu.VMEM((2, SUBN, SUB), jnp.int32),
            pltpu.VMEM((2, SUBN, SUB), jnp.int32),
            pltpu.VMEM((C * D // 2,), jnp.int32),
            pltpu.VMEM((C * D // 2,), jnp.int32),
            pltpu.VMEM((2, C, D), jnp.float32),
            pltpu.VMEM((BPT,), jnp.int32),
            pltpu.SemaphoreType.DMA,
            pltpu.SemaphoreType.DMA,
            pltpu.SemaphoreType.DMA,
            pltpu.SemaphoreType.DMA,
            pltpu.SemaphoreType.DMA,
            pltpu.SemaphoreType.DMA,
        ],
    )
    return f(xa, eb, src, dst, idx, x, zeros)


def _upd_body(xs_ref, a0_ref, a1_ref, w_ref, b_ref, o_ref):
    hcat = jnp.concatenate([xs_ref[...], a0_ref[...] + a1_ref[...]], axis=1)
    o_ref[...] = jnp.maximum(
        jnp.dot(hcat, w_ref[...], preferred_element_type=jnp.float32)
        + b_ref[...], 0.0)


def _tc_post(xsel, aggsel, W_upd, b_upd):
    return pl.pallas_call(
        _upd_body,
        out_shape=jax.ShapeDtypeStruct((B, D), jnp.float32),
    )(xsel, aggsel[0], aggsel[1], W_upd, b_upd.reshape(1, D))


_PLO = (np.arange(D // 32)[:, None] * 32 + np.arange(16)[None, :]).reshape(-1)
_PERM = np.concatenate([_PLO, _PLO + 16])


def kernel(x, edge_index, edge_attr, idx, W_msg, b_msg, W_upd, b_upd):
    src = edge_index[0].reshape(E // C, SUBN, SUB)
    dst = edge_index[1].reshape(E // C, SUBN, SUB)
    W1 = W_msg[:D]
    W2 = W_msg[D:][:, _PERM]
    xa, eb = _tc_pre(x, edge_attr, W1, W2, b_msg[_PERM])
    zeros = jnp.zeros((RPT, D), jnp.float32)
    aggsel, xsel = _sc_call(xa, eb.reshape(E * D // 2), src, dst, idx, x,
                            zeros)
    return _tc_post(xsel, aggsel, W_upd, b_upd)


# speedup vs baseline: 1.2176x; 1.2176x over previous
import functools

import numpy as np

import jax
import jax.numpy as jnp
from jax import lax
from jax.experimental import pallas as pl
from jax.experimental.pallas import tpu as pltpu
from jax.experimental.pallas import tpu_sc as plsc

N = 10000
E = 320000
D = 128
DE = 16
B = 1024

NC = 2
NS = 16
NW = NC * NS
EPW = E // NW
SUB = 80
SUBN = 1
C = SUB * SUBN
NCHUNK = EPW // C
RPT = N // NS
BPT = B // NS


def _xa_body(x_ref, w_ref, o_ref):
    o_ref[...] = jnp.dot(x_ref[...], w_ref[...],
                         preferred_element_type=jnp.float32)


def _eb_body(a_ref, w_ref, b_ref, o_ref):
    m = (jnp.dot(a_ref[...], w_ref[...],
                 preferred_element_type=jnp.float32) + b_ref[...])
    bits = jax.lax.bitcast_convert_type(m, jnp.int32) + jnp.int32(0x8000)
    lo = jax.lax.shift_right_logical(bits[:, :D // 2], 16)
    hi = jnp.bitwise_and(bits[:, D // 2:], jnp.int32(-65536))
    o_ref[...] = jnp.bitwise_or(lo, hi)


_EB_R = 8000


def _tc_pre(x, attr, W1, W2, b_msg):
    xa = pl.pallas_call(
        _xa_body,
        out_shape=jax.ShapeDtypeStruct((N, D), jnp.float32),
    )(x, W1)
    eb = pl.pallas_call(
        _eb_body,
        grid=(E // _EB_R,),
        in_specs=[
            pl.BlockSpec((_EB_R, DE), lambda i: (i, 0)),
            pl.BlockSpec((DE, D), lambda i: (0, 0)),
            pl.BlockSpec((1, D), lambda i: (0, 0)),
        ],
        out_specs=pl.BlockSpec((_EB_R, D // 2), lambda i: (i, 0)),
        out_shape=jax.ShapeDtypeStruct((E, D // 2), jnp.int32),
    )(attr, W2, b_msg.reshape(1, D))
    return xa, eb


def _sc_body(xa_hbm, eb_hbm, src_hbm, dst_hbm, idx_hbm, x_hbm, zeros_hbm,
             aggsel_hbm, xsel_hbm,
             agg_sh, src_v, dst_v, e_v, g_v, idx_v,
             sem_l0, sem_l1, sem_g0, sem_g1, sem_s0, sem_s1):
    cid = lax.axis_index("c")
    sid = lax.axis_index("s")
    wid = cid * NS + sid
    sem_l = (sem_l0, sem_l1)
    sem_g = (sem_g0, sem_g1)
    sem_s = (sem_s0, sem_s1)

    def issue_loads(k, s):
        crow = wid * NCHUNK + k
        ebase = wid * EPW + k * C
        pltpu.async_copy(src_hbm.at[crow], src_v.at[s], sem_l[s])
        pltpu.async_copy(dst_hbm.at[crow], dst_v.at[s], sem_l[s])
        eoff = pl.multiple_of(ebase, C)
        pltpu.async_copy(eb_hbm.at[pl.ds(eoff, C)], e_v.at[s], sem_l[s])

    def wait_loads(s):
        pltpu.make_async_copy(src_hbm.at[0], src_v.at[s],
                              sem_l[s]).wait()
        pltpu.make_async_copy(dst_hbm.at[0], dst_v.at[s],
                              sem_l[s]).wait()
        pltpu.make_async_copy(eb_hbm.at[pl.ds(0, C)], e_v.at[s],
                              sem_l[s]).wait()

    def issue_gather(s):
        for j in range(SUBN):
            pltpu.async_copy(xa_hbm.at[src_v.at[s, j]],
                             g_v.at[s, pl.ds(j * SUB, SUB)], sem_g[s])

    def wait_gather(s):
        pltpu.make_async_copy(xa_hbm.at[pl.ds(0, C)], g_v.at[s],
                              sem_g[s]).wait()

    def issue_scatter(s):
        for j in range(SUBN):
            pltpu.async_copy(g_v.at[s, pl.ds(j * SUB, SUB)],
                             agg_sh.at[dst_v.at[s, j]], sem_s[s], add=True)

    def wait_scatter(s):
        pltpu.make_async_copy(xa_hbm.at[pl.ds(0, C)], g_v.at[s],
                              sem_s[s]).wait()

    issue_loads(0, 0)
    issue_loads(1, 1)
    pltpu.sync_copy(zeros_hbm, agg_sh.at[pl.ds(sid * RPT, RPT)])
    plsc.subcore_barrier()

    wait_loads(0)
    issue_gather(0)

    def pipeline_step(i, b):
        o = 1 - b

        @pl.when(i + 1 < NCHUNK)
        def _():
            wait_loads(o)

            @pl.when(i >= 1)
            def _():
                wait_scatter(o)

            issue_gather(o)

        wait_gather(b)

        @plsc.parallel_loop(0, C, step=1, unroll=4)
        def _(r):
            for k2 in range(D // 32):
                w = e_v[b, r, pl.ds(16 * k2, 16)]
                ea = plsc.bitcast(jnp.left_shift(w, 16), jnp.float32)
                eo = plsc.bitcast(jnp.bitwise_and(w, jnp.int32(-65536)),
                                  jnp.float32)
                sl0 = pl.ds(32 * k2, 16)
                sl1 = pl.ds(32 * k2 + 16, 16)
                g_v[b, r, sl0] = jnp.maximum(g_v[b, r, sl0] + ea, 0.0)
                g_v[b, r, sl1] = jnp.maximum(g_v[b, r, sl1] + eo, 0.0)

        issue_scatter(b)

        @pl.when(i + 2 < NCHUNK)
        def _():
            issue_loads(i + 2, b)

    def outer(t, carry):
        pipeline_step(2 * t, 0)
        pipeline_step(2 * t + 1, 1)
        return carry

    lax.fori_loop(0, NCHUNK // 2, outer, 0)
    pipeline_step(jnp.int32(NCHUNK - 1), 0)
    wait_scatter(0)
    wait_scatter(1)
    plsc.subcore_barrier()

    pltpu.sync_copy(idx_hbm.at[pl.ds(sid * BPT, BPT)], idx_v)
    pltpu.async_copy(agg_sh.at[idx_v], g_v.at[0, pl.ds(0, BPT)], sem_g0).wait()
    pltpu.sync_copy(g_v.at[0, pl.ds(0, BPT)],
                    aggsel_hbm.at[cid, pl.ds(sid * BPT, BPT)])

    @pl.when(cid == 0)
    def _():
        pltpu.async_copy(x_hbm.at[idx_v], g_v.at[1, pl.ds(0, BPT)],
                         sem_g1).wait()
        pltpu.sync_copy(g_v.at[1, pl.ds(0, BPT)],
                        xsel_hbm.at[pl.ds(sid * BPT, BPT)])


def _sc_call(xa, eb, src, dst, idx, x, zeros):
    mesh = plsc.VectorSubcoreMesh(core_axis_name="c", subcore_axis_name="s",
                                  num_cores=NC, num_subcores=NS)
    f = pl.kernel(
        _sc_body,
        out_type=(jax.ShapeDtypeStruct((NC, B, D), jnp.float32),
                  jax.ShapeDtypeStruct((B, D), jnp.float32)),
        mesh=mesh,
        compiler_params=pltpu.CompilerParams(needs_layout_passes=False),
        scratch_types=[
            pltpu.VMEM_SHARED((N, D), jnp.float32),
            pltpu.VMEM((2, SUBN, SUB), jnp.int32),
            pltpu.VMEM((2, SUBN, SUB), jnp.int32),
            pltpu.VMEM((2, C, D // 2), jnp.int32),
            pltpu.VMEM((2, C, D), jnp.float32),
            pltpu.VMEM((BPT,), jnp.int32),
            pltpu.SemaphoreType.DMA,
            pltpu.SemaphoreType.DMA,
            pltpu.SemaphoreType.DMA,
            pltpu.SemaphoreType.DMA,
            pltpu.SemaphoreType.DMA,
            pltpu.SemaphoreType.DMA,
        ],
    )
    return f(xa, eb, src, dst, idx, x, zeros)


def _upd_body(xs_ref, a0_ref, a1_ref, w_ref, b_ref, o_ref):
    hcat = jnp.concatenate([xs_ref[...], a0_ref[...] + a1_ref[...]], axis=1)
    o_ref[...] = jnp.maximum(
        jnp.dot(hcat, w_ref[...], preferred_element_type=jnp.float32)
        + b_ref[...], 0.0)


def _tc_post(xsel, aggsel, W_upd, b_upd):
    return pl.pallas_call(
        _upd_body,
        out_shape=jax.ShapeDtypeStruct((B, D), jnp.float32),
    )(xsel, aggsel[0], aggsel[1], W_upd, b_upd.reshape(1, D))


_PLO = (np.arange(D // 32)[:, None] * 32 + np.arange(16)[None, :]).reshape(-1)
_PERM = np.concatenate([_PLO, _PLO + 16])


def kernel(x, edge_index, edge_attr, idx, W_msg, b_msg, W_upd, b_upd):
    src = edge_index[0].reshape(E // C, SUBN, SUB)
    dst = edge_index[1].reshape(E // C, SUBN, SUB)
    W1 = W_msg[:D]
    W2 = W_msg[D:][:, _PERM]
    xa, eb = _tc_pre(x, edge_attr, W1, W2, b_msg[_PERM])
    zeros = jnp.zeros((RPT, D), jnp.float32)
    aggsel, xsel = _sc_call(xa, eb, src, dst, idx, x, zeros)
    return _tc_post(xsel, aggsel, W_upd, b_upd)


# speedup vs baseline: 1.2240x; 1.0052x over previous
import functools

import numpy as np

import jax
import jax.numpy as jnp
from jax import lax
from jax.experimental import pallas as pl
from jax.experimental.pallas import tpu as pltpu
from jax.experimental.pallas import tpu_sc as plsc

N = 10000
E = 320000
D = 128
DE = 16
B = 1024

NC = 2
NS = 16
NW = NC * NS
EPW = E // NW
SUB = 80
SUBN = 1
C = SUB * SUBN
NCHUNK = EPW // C
RPT = N // NS
BPT = B // NS


def _xa_body(x_ref, w_ref, o_ref):
    o_ref[...] = jnp.dot(x_ref[...], w_ref[...],
                         preferred_element_type=jnp.float32)


def _eb_body(a_ref, w_ref, b_ref, o_ref):
    m = (jnp.dot(a_ref[...], w_ref[...],
                 preferred_element_type=jnp.float32) + b_ref[...])
    bits = jax.lax.bitcast_convert_type(m, jnp.int32) + jnp.int32(0x8000)
    lo = jax.lax.shift_right_logical(bits[:, :D // 2], 16)
    hi = jnp.bitwise_and(bits[:, D // 2:], jnp.int32(-65536))
    o_ref[...] = jnp.bitwise_or(lo, hi)


_EB_R = 16000


def _tc_pre(x, attr, W1, W2, b_msg):
    xa = pl.pallas_call(
        _xa_body,
        out_shape=jax.ShapeDtypeStruct((N, D), jnp.float32),
    )(x, W1)
    eb = pl.pallas_call(
        _eb_body,
        grid=(E // _EB_R,),
        in_specs=[
            pl.BlockSpec((_EB_R, DE), lambda i: (i, 0)),
            pl.BlockSpec((DE, D), lambda i: (0, 0)),
            pl.BlockSpec((1, D), lambda i: (0, 0)),
        ],
        out_specs=pl.BlockSpec((_EB_R, D // 2), lambda i: (i, 0)),
        out_shape=jax.ShapeDtypeStruct((E, D // 2), jnp.int32),
    )(attr, W2, b_msg.reshape(1, D))
    return xa, eb


def _sc_body(xa_hbm, eb_hbm, src_hbm, dst_hbm, idx_hbm, x_hbm, zeros_hbm,
             aggsel_hbm, xsel_hbm,
             agg_sh, src_v, dst_v, e_v, g_v, idx_v,
             sem_l0, sem_l1, sem_g0, sem_g1, sem_s0, sem_s1):
    cid = lax.axis_index("c")
    sid = lax.axis_index("s")
    wid = cid * NS + sid
    sem_l = (sem_l0, sem_l1)
    sem_g = (sem_g0, sem_g1)
    sem_s = (sem_s0, sem_s1)

    def issue_loads(k, s):
        crow = wid * NCHUNK + k
        ebase = wid * EPW + k * C
        pltpu.async_copy(src_hbm.at[crow], src_v.at[s], sem_l[s])
        pltpu.async_copy(dst_hbm.at[crow], dst_v.at[s], sem_l[s])
        eoff = pl.multiple_of(ebase, C)
        pltpu.async_copy(eb_hbm.at[pl.ds(eoff, C)], e_v.at[s], sem_l[s])

    def wait_loads(s):
        pltpu.make_async_copy(src_hbm.at[0], src_v.at[s],
                              sem_l[s]).wait()
        pltpu.make_async_copy(dst_hbm.at[0], dst_v.at[s],
                              sem_l[s]).wait()
        pltpu.make_async_copy(eb_hbm.at[pl.ds(0, C)], e_v.at[s],
                              sem_l[s]).wait()

    def issue_gather(s):
        for j in range(SUBN):
            pltpu.async_copy(xa_hbm.at[src_v.at[s, j]],
                             g_v.at[s, pl.ds(j * SUB, SUB)], sem_g[s])

    def wait_gather(s):
        pltpu.make_async_copy(xa_hbm.at[pl.ds(0, C)], g_v.at[s],
                              sem_g[s]).wait()

    def issue_scatter(s):
        for j in range(SUBN):
            pltpu.async_copy(g_v.at[s, pl.ds(j * SUB, SUB)],
                             agg_sh.at[dst_v.at[s, j]], sem_s[s], add=True)

    def wait_scatter(s):
        pltpu.make_async_copy(xa_hbm.at[pl.ds(0, C)], g_v.at[s],
                              sem_s[s]).wait()

    issue_loads(0, 0)
    issue_loads(1, 1)
    pltpu.sync_copy(zeros_hbm, agg_sh.at[pl.ds(sid * RPT, RPT)])
    plsc.subcore_barrier()

    wait_loads(0)
    issue_gather(0)

    def pipeline_step(i, b):
        o = 1 - b

        @pl.when(i + 1 < NCHUNK)
        def _():
            wait_loads(o)

            @pl.when(i >= 1)
            def _():
                wait_scatter(o)

            issue_gather(o)

        wait_gather(b)

        @plsc.parallel_loop(0, C, step=1, unroll=4)
        def _(r):
            for k2 in range(D // 32):
                w = e_v[b, r, pl.ds(16 * k2, 16)]
                ea = plsc.bitcast(jnp.left_shift(w, 16), jnp.float32)
                eo = plsc.bitcast(jnp.bitwise_and(w, jnp.int32(-65536)),
                                  jnp.float32)
                sl0 = pl.ds(32 * k2, 16)
                sl1 = pl.ds(32 * k2 + 16, 16)
                g_v[b, r, sl0] = jnp.maximum(g_v[b, r, sl0] + ea, 0.0)
                g_v[b, r, sl1] = jnp.maximum(g_v[b, r, sl1] + eo, 0.0)

        issue_scatter(b)

        @pl.when(i + 2 < NCHUNK)
        def _():
            issue_loads(i + 2, b)

    def outer(t, carry):
        pipeline_step(2 * t, 0)
        pipeline_step(2 * t + 1, 1)
        return carry

    lax.fori_loop(0, NCHUNK // 2, outer, 0)
    pipeline_step(jnp.int32(NCHUNK - 1), 0)
    wait_scatter(0)
    wait_scatter(1)
    plsc.subcore_barrier()

    pltpu.sync_copy(idx_hbm.at[pl.ds(sid * BPT, BPT)], idx_v)
    pltpu.async_copy(agg_sh.at[idx_v], g_v.at[0, pl.ds(0, BPT)], sem_g0).wait()
    pltpu.sync_copy(g_v.at[0, pl.ds(0, BPT)],
                    aggsel_hbm.at[cid, pl.ds(sid * BPT, BPT)])

    @pl.when(cid == 0)
    def _():
        pltpu.async_copy(x_hbm.at[idx_v], g_v.at[1, pl.ds(0, BPT)],
                         sem_g1).wait()
        pltpu.sync_copy(g_v.at[1, pl.ds(0, BPT)],
                        xsel_hbm.at[pl.ds(sid * BPT, BPT)])


def _sc_call(xa, eb, src, dst, idx, x, zeros):
    mesh = plsc.VectorSubcoreMesh(core_axis_name="c", subcore_axis_name="s",
                                  num_cores=NC, num_subcores=NS)
    f = pl.kernel(
        _sc_body,
        out_type=(jax.ShapeDtypeStruct((NC, B, D), jnp.float32),
                  jax.ShapeDtypeStruct((B, D), jnp.float32)),
        mesh=mesh,
        compiler_params=pltpu.CompilerParams(needs_layout_passes=False),
        scratch_types=[
            pltpu.VMEM_SHARED((N, D), jnp.float32),
            pltpu.VMEM((2, SUBN, SUB), jnp.int32),
            pltpu.VMEM((2, SUBN, SUB), jnp.int32),
            pltpu.VMEM((2, C, D // 2), jnp.int32),
            pltpu.VMEM((2, C, D), jnp.float32),
            pltpu.VMEM((BPT,), jnp.int32),
            pltpu.SemaphoreType.DMA,
            pltpu.SemaphoreType.DMA,
            pltpu.SemaphoreType.DMA,
            pltpu.SemaphoreType.DMA,
            pltpu.SemaphoreType.DMA,
            pltpu.SemaphoreType.DMA,
        ],
    )
    return f(xa, eb, src, dst, idx, x, zeros)


def _upd_body(xs_ref, a0_ref, a1_ref, w_ref, b_ref, o_ref):
    hcat = jnp.concatenate([xs_ref[...], a0_ref[...] + a1_ref[...]], axis=1)
    o_ref[...] = jnp.maximum(
        jnp.dot(hcat, w_ref[...], preferred_element_type=jnp.float32)
        + b_ref[...], 0.0)


def _tc_post(xsel, aggsel, W_upd, b_upd):
    return pl.pallas_call(
        _upd_body,
        out_shape=jax.ShapeDtypeStruct((B, D), jnp.float32),
    )(xsel, aggsel[0], aggsel[1], W_upd, b_upd.reshape(1, D))


_PLO = (np.arange(D // 32)[:, None] * 32 + np.arange(16)[None, :]).reshape(-1)
_PERM = np.concatenate([_PLO, _PLO + 16])


def kernel(x, edge_index, edge_attr, idx, W_msg, b_msg, W_upd, b_upd):
    src = edge_index[0].reshape(E // C, SUBN, SUB)
    dst = edge_index[1].reshape(E // C, SUBN, SUB)
    W1 = W_msg[:D]
    W2 = W_msg[D:][:, _PERM]
    xa, eb = _tc_pre(x, edge_attr, W1, W2, b_msg[_PERM])
    zeros = jnp.zeros((RPT, D), jnp.float32)
    aggsel, xsel = _sc_call(xa, eb, src, dst, idx, x, zeros)
    return _tc_post(xsel, aggsel, W_upd, b_upd)
